# baseline (speedup 1.0000x reference)
import functools

import jax
import jax.numpy as jnp
from jax import lax
from jax.experimental import pallas as pl
from jax.experimental.pallas import tpu as pltpu
from jax.experimental.pallas import tpu_sc as plsc

NC = 2
NS = 16
NW = NC * NS
CHUNK = 128


def _matmul(x, W, b):
    n, d = x.shape
    blk = 400
    grid = n // blk

    def body(x_ref, w_ref, b_ref, o_ref):
        o_ref[...] = lax.dot_general(
            x_ref[...], w_ref[...],
            (((1,), (1,)), ((), ())),
            preferred_element_type=jnp.float32,
            precision=lax.Precision.HIGHEST,
        ) + b_ref[...]

    return pl.pallas_call(
        body,
        grid=(grid,),
        in_specs=[
            pl.BlockSpec((blk, d), lambda i: (i, 0)),
            pl.BlockSpec((d, d), lambda i: (0, 0)),
            pl.BlockSpec((1, d), lambda i: (0, 0)),
        ],
        out_specs=pl.BlockSpec((blk, d), lambda i: (i, 0)),
        out_shape=jax.ShapeDtypeStruct((n, d), jnp.float32),
    )(x, W, b.reshape(1, d))


def _sc_aggregate(h, src_w, dst_w, zeros_blk, acc_rows, rpt, k_chunks):
    n, d = h.shape[1:]
    mesh = plsc.VectorSubcoreMesh(
        core_axis_name="c", subcore_axis_name="s",
        num_cores=NC, num_subcores=NS)

    @functools.partial(
        pl.kernel,
        out_type=jax.ShapeDtypeStruct((NC, acc_rows, d), jnp.float32),
        mesh=mesh,
        scratch_types=[
            pltpu.VMEM((k_chunks, CHUNK), jnp.int32),
            pltpu.VMEM((k_chunks, CHUNK), jnp.int32),
            pltpu.VMEM((CHUNK, d), jnp.float32),
            pltpu.VMEM_SHARED((acc_rows, d), jnp.float32),
            pltpu.SemaphoreType.DMA,
        ],
    )
    def k(h_hbm, src_hbm, dst_hbm, z_hbm, out_hbm,
          src_v, dst_v, buf, acc, sem):
        c = lax.axis_index("c")
        h_mine = h_hbm.at[c]
        s = lax.axis_index("s")
        w = c * NS + s
        zero_cp = pltpu.async_copy(z_hbm, acc.at[pl.ds(s * rpt, rpt)], sem)
        src_cp = pltpu.async_copy(src_hbm.at[w], src_v, sem)
        dst_cp = pltpu.async_copy(dst_hbm.at[w], dst_v, sem)
        zero_cp.wait()
        src_cp.wait()
        dst_cp.wait()
        plsc.subcore_barrier()

        @pl.loop(0, k_chunks)
        def _(j):
            pltpu.async_copy(h_mine.at[src_v.at[j]], buf, sem).wait()
            pltpu.sync_copy(buf, acc.at[dst_v.at[j]], add=True)

        plsc.subcore_barrier()
        pltpu.sync_copy(acc.at[pl.ds(s * rpt, rpt)],
                        out_hbm.at[c, pl.ds(s * rpt, rpt)])

    return k(h, src_w, dst_w, zeros_blk)


def _combine(partials, h):
    n, d = h.shape
    blk = 400
    grid = n // blk

    def body(p_ref, h_ref, o_ref):
        o_ref[...] = jnp.maximum(p_ref[0] + p_ref[1] + h_ref[...], 0.0)

    return pl.pallas_call(
        body,
        grid=(grid,),
        in_specs=[
            pl.BlockSpec((NC, blk, d), lambda i: (0, i, 0)),
            pl.BlockSpec((blk, d), lambda i: (i, 0)),
        ],
        out_specs=pl.BlockSpec((blk, d), lambda i: (i, 0)),
        out_shape=jax.ShapeDtypeStruct((n, d), jnp.float32),
    )(partials, h)


def kernel(node_feats, edge_index, W, b):
    n, d = node_feats.shape
    e = edge_index.shape[1]

    rpt = (-(-(n + 1) // NS) + 63) // 64 * 64
    acc_rows = NS * rpt

    k_chunks = -(-e // (NW * CHUNK))
    e_pad = NW * k_chunks * CHUNK
    src = jnp.concatenate(
        [edge_index[0], jnp.zeros((e_pad - e,), jnp.int32)]).reshape(
            NW, k_chunks, CHUNK)
    dst = jnp.concatenate(
        [edge_index[1], jnp.full((e_pad - e,), n, jnp.int32)]).reshape(
            NW, k_chunks, CHUNK)
    zeros_blk = jnp.zeros((rpt, d), jnp.float32)

    h = _matmul(node_feats, W, b)
    hs = jnp.stack([h, h])
    partials = _sc_aggregate(hs, src, dst, zeros_blk, acc_rows, rpt, k_chunks)
    return _combine(partials, h)


# speedup vs baseline: 1.0794x; 1.0794x over previous
import functools

import jax
import jax.numpy as jnp
from jax import lax
from jax.experimental import pallas as pl
from jax.experimental.pallas import tpu as pltpu
from jax.experimental.pallas import tpu_sc as plsc

NC = 2
NS = 16
NW = NC * NS
CHUNK = 128


def _matmul(x, W, b):
    n, d = x.shape
    blk = 400
    grid = n // blk

    def body(x_ref, w_ref, b_ref, o_ref):
        o_ref[...] = lax.dot_general(
            x_ref[...], w_ref[...],
            (((1,), (1,)), ((), ())),
            preferred_element_type=jnp.float32,
            precision=lax.Precision.HIGHEST,
        ) + b_ref[...]

    return pl.pallas_call(
        body,
        grid=(grid,),
        in_specs=[
            pl.BlockSpec((blk, d), lambda i: (i, 0)),
            pl.BlockSpec((d, d), lambda i: (0, 0)),
            pl.BlockSpec((1, d), lambda i: (0, 0)),
        ],
        out_specs=pl.BlockSpec((blk, d), lambda i: (i, 0)),
        out_shape=jax.ShapeDtypeStruct((n, d), jnp.float32),
    )(x, W, b.reshape(1, d))


def _sc_aggregate(h, src_w, dst_w, zeros_blk, acc_rows, rpt, k_chunks):
    n, d = h.shape
    mesh = plsc.VectorSubcoreMesh(
        core_axis_name="c", subcore_axis_name="s",
        num_cores=NC, num_subcores=NS)

    @functools.partial(
        pl.kernel,
        out_type=jax.ShapeDtypeStruct((NC, acc_rows, d), jnp.float32),
        mesh=mesh,
        scratch_types=[
            pltpu.VMEM((k_chunks, CHUNK), jnp.int32),
            pltpu.VMEM((k_chunks, CHUNK), jnp.int32),
            pltpu.VMEM((CHUNK, d), jnp.float32),
            pltpu.VMEM_SHARED((acc_rows, d), jnp.float32),
            pltpu.SemaphoreType.DMA,
            [pltpu.SemaphoreType.DMA] * 3,
        ],
    )
    def k(h_hbm, src_hbm, dst_hbm, z_hbm, out_hbm,
          src_v, dst_v, buf, acc, sem, init_sems):
        c = lax.axis_index("c")
        s = lax.axis_index("s")
        w = c * NS + s
        zero_cp = pltpu.async_copy(
            z_hbm, acc.at[pl.ds(s * rpt, rpt)], init_sems[0])
        src_cp = pltpu.async_copy(src_hbm.at[w], src_v, init_sems[1])
        dst_cp = pltpu.async_copy(dst_hbm.at[w], dst_v, init_sems[2])
        zero_cp.wait()
        src_cp.wait()
        dst_cp.wait()
        plsc.subcore_barrier()

        @pl.loop(0, k_chunks)
        def _(j):
            pltpu.async_copy(h_hbm.at[src_v.at[j]], buf, sem).wait()
            pltpu.sync_copy(buf, acc.at[dst_v.at[j]], add=True)

        plsc.subcore_barrier()
        pltpu.sync_copy(acc.at[pl.ds(s * rpt, rpt)],
                        out_hbm.at[c, pl.ds(s * rpt, rpt)])

    return k(h, src_w, dst_w, zeros_blk)


def _combine(partials, h):
    n, d = h.shape
    blk = 400
    grid = n // blk

    def body(p_ref, h_ref, o_ref):
        o_ref[...] = jnp.maximum(p_ref[0] + p_ref[1] + h_ref[...], 0.0)

    return pl.pallas_call(
        body,
        grid=(grid,),
        in_specs=[
            pl.BlockSpec((NC, blk, d), lambda i: (0, i, 0)),
            pl.BlockSpec((blk, d), lambda i: (i, 0)),
        ],
        out_specs=pl.BlockSpec((blk, d), lambda i: (i, 0)),
        out_shape=jax.ShapeDtypeStruct((n, d), jnp.float32),
    )(partials, h)


def kernel(node_feats, edge_index, W, b):
    n, d = node_feats.shape
    e = edge_index.shape[1]

    rpt = (-(-(n + 1) // NS) + 63) // 64 * 64
    acc_rows = NS * rpt

    k_chunks = -(-e // (NW * CHUNK))
    e_pad = NW * k_chunks * CHUNK
    src = jnp.concatenate(
        [edge_index[0], jnp.zeros((e_pad - e,), jnp.int32)]).reshape(
            NW, k_chunks, CHUNK)
    dst = jnp.concatenate(
        [edge_index[1], jnp.full((e_pad - e,), n, jnp.int32)]).reshape(
            NW, k_chunks, CHUNK)
    zeros_blk = jnp.zeros((rpt, d), jnp.float32)

    h = _matmul(node_feats, W, b)
    partials = _sc_aggregate(h, src, dst, zeros_blk, acc_rows, rpt, k_chunks)
    return _combine(partials, h)
